# baseline (speedup 1.0000x reference)
import functools

import jax
import jax.numpy as jnp
from jax.experimental import pallas as pl

_HID = 256
_NUM_HEADS = 4
_L_MOL = 2


def _rup(v, m):
    return (v + m - 1) // m * m


def _mm_body(act, has_bias, *refs):
    if has_bias:
        a_ref, b_ref, bias_ref, o_ref = refs
    else:
        a_ref, b_ref, o_ref = refs
    y = jnp.dot(a_ref[...], b_ref[...], preferred_element_type=jnp.float32)
    if has_bias:
        y = y + bias_ref[...]
    if act == 'leaky':
        y = jnp.where(y >= 0.0, y, 0.01 * y)
    elif act == 'elu':
        y = jnp.where(y >= 0.0, y, jnp.expm1(y))
    o_ref[...] = y


def _mm(a, b, bias=None, act=None, tile_m=512):
    m, k = a.shape
    _, n = b.shape
    kp = _rup(k, 128)
    np_ = _rup(n, 128)
    tile_m = min(tile_m, _rup(m, 8))
    mp = _rup(m, tile_m)
    a = jnp.pad(a, ((0, mp - m), (0, kp - k)))
    b = jnp.pad(b, ((0, kp - k), (0, np_ - n)))
    has_bias = bias is not None
    operands = [a, b]
    in_specs = [
        pl.BlockSpec((tile_m, kp), lambda i: (i, 0)),
        pl.BlockSpec((kp, np_), lambda i: (0, 0)),
    ]
    if has_bias:
        operands.append(jnp.pad(bias.reshape(1, -1), ((0, 0), (0, np_ - n))))
        in_specs.append(pl.BlockSpec((1, np_), lambda i: (0, 0)))
    out = pl.pallas_call(
        functools.partial(_mm_body, act, has_bias),
        grid=(mp // tile_m,),
        in_specs=in_specs,
        out_specs=pl.BlockSpec((tile_m, np_), lambda i: (i, 0)),
        out_shape=jax.ShapeDtypeStruct((mp, np_), jnp.float32),
    )(*operands)
    return out[:m, :n]


def _gru_body(x_ref, h_ref, wi_ref, wh_ref, bi_ref, bh_ref, o_ref):
    gi = jnp.dot(x_ref[...], wi_ref[...],
                 preferred_element_type=jnp.float32) + bi_ref[...]
    gh = jnp.dot(h_ref[...], wh_ref[...],
                 preferred_element_type=jnp.float32) + bh_ref[...]
    d = _HID
    r = jax.nn.sigmoid(gi[:, :d] + gh[:, :d])
    z = jax.nn.sigmoid(gi[:, d:2 * d] + gh[:, d:2 * d])
    n = jnp.tanh(gi[:, 2 * d:] + r * gh[:, 2 * d:])
    o_ref[...] = (1.0 - z) * n + z * h_ref[...]


def _gru(xin, h, wi, wh, bi, bh, tile_m=512):
    m = xin.shape[0]
    tile_m = min(tile_m, _rup(m, 8))
    mp = _rup(m, tile_m)
    xp = jnp.pad(xin, ((0, mp - m), (0, 0)))
    hp = jnp.pad(h, ((0, mp - m), (0, 0)))
    out = pl.pallas_call(
        _gru_body,
        grid=(mp // tile_m,),
        in_specs=[
            pl.BlockSpec((tile_m, _HID), lambda i: (i, 0)),
            pl.BlockSpec((tile_m, _HID), lambda i: (i, 0)),
            pl.BlockSpec((_HID, 3 * _HID), lambda i: (0, 0)),
            pl.BlockSpec((_HID, 3 * _HID), lambda i: (0, 0)),
            pl.BlockSpec((1, 3 * _HID), lambda i: (0, 0)),
            pl.BlockSpec((1, 3 * _HID), lambda i: (0, 0)),
        ],
        out_specs=pl.BlockSpec((tile_m, _HID), lambda i: (i, 0)),
        out_shape=jax.ShapeDtypeStruct((mp, _HID), jnp.float32),
    )(xp, hp, wi, wh, bi.reshape(1, -1), bh.reshape(1, -1))
    return out[:m]


def _seg_softmax(logits, segs, num):
    m = jax.ops.segment_max(logits, segs, num_segments=num)
    m = jnp.where(jnp.isfinite(m), m, 0.0)
    ex = jnp.exp(logits - m[segs])
    den = jax.ops.segment_sum(ex, segs, num_segments=num)
    return ex / (den[segs] + 1e-16)


def _head(x, edge_attr, src, dst, batch, p, n_nodes, n_graphs):
    h = _mm(x, p['lin1_W'], bias=p['lin1_b'], act='leaky')
    cat = jnp.concatenate([h[src], edge_attr], axis=1)
    xj = _mm(cat, p['ge_W'], bias=p['ge_b'], act='leaky')
    xal = _mm(xj, p['ge_al'][:, None])[:, 0]
    har = _mm(h, p['ge_ar'][:, None])[:, 0]
    alpha = jax.nn.leaky_relu(xal + har[dst])
    alpha = _seg_softmax(alpha, dst, n_nodes)
    xw2 = _mm(xj, p['ge_W2'])
    msg = jax.ops.segment_sum(alpha[:, None] * xw2, dst, num_segments=n_nodes)
    msg = jax.nn.elu(msg)
    h = _gru(msg, h, p['gru1_Wi'], p['gru1_Wh'], p['gru1_bi'], p['gru1_bh'])

    hw = _mm(h, p['gat_W'])
    asd = _mm(hw, jnp.stack([p['gat_as'], p['gat_ad']], axis=1))
    alpha = jax.nn.leaky_relu(asd[src, 0] + asd[dst, 1])
    alpha = _seg_softmax(alpha, dst, n_nodes)
    msg = jax.ops.segment_sum(alpha[:, None] * hw[src], dst,
                              num_segments=n_nodes)
    msg = jax.nn.elu(msg)
    h = _gru(msg, h, p['gru2_Wi'], p['gru2_Wh'], p['gru2_bi'], p['gru2_bh'])

    g = jax.nn.relu(jax.ops.segment_sum(h, batch, num_segments=n_graphs))
    hw = _mm(h, p['mol_W'])
    hw_as = _mm(hw, p['mol_as'][:, None])[:, 0]
    for _ in range(_L_MOL):
        g_ad = _mm(g, p['mol_ad'][:, None])[:, 0]
        alpha = jax.nn.leaky_relu(hw_as + g_ad[batch])
        alpha = _seg_softmax(alpha, batch, n_graphs)
        msg = jax.ops.segment_sum(alpha[:, None] * hw, batch,
                                  num_segments=n_graphs)
        msg = jax.nn.elu(msg)
        g = _gru(msg, g, p['grum_Wi'], p['grum_Wh'], p['grum_bi'],
                 p['grum_bh'])
    return _mm(g, p['lin2_W'], bias=p['lin2_b'])


_HEAD_KEYS = ('lin1', 'ge', 'gru', 'gat', 'mol', 'lin2')


@jax.jit
def _forward_impl(x, edge_index, edge_attr, batch, params):
    n_nodes = x.shape[0]
    n_graphs = params['fa_W'].shape[1]
    src, dst = edge_index[0], edge_index[1]
    outs = []
    for hi in range(_NUM_HEADS):
        ph = {k: v[hi] for k, v in params.items()
              if k.startswith(_HEAD_KEYS)}
        outs.append(_head(x, edge_attr, src, dst, batch, ph, n_nodes, 256))
    cat = jnp.concatenate(outs, axis=-1)
    y = _mm(cat, params['fa_W'], bias=params['fa_b'])
    mu = y.mean(axis=0)
    var = y.var(axis=0)
    y = (y - mu) / jnp.sqrt(var + 1e-5) * params['fa_gamma'] + params['fa_beta']
    return jax.nn.relu(y)


def kernel(x, edge_index, edge_attr, batch, params):
    return _forward_impl(x, edge_index, edge_attr, batch, params)


# speedup vs baseline: 1.0621x; 1.0621x over previous
import functools

import jax
import jax.numpy as jnp
from jax.experimental import pallas as pl

_HID = 256
_NUM_HEADS = 4
_L_MOL = 2


def _rup(v, m):
    return (v + m - 1) // m * m


def _mm_body(act, has_bias, *refs):
    if has_bias:
        a_ref, b_ref, bias_ref, o_ref = refs
    else:
        a_ref, b_ref, o_ref = refs
    y = jnp.dot(a_ref[...], b_ref[...], preferred_element_type=jnp.float32)
    if has_bias:
        y = y + bias_ref[...]
    if act == 'leaky':
        y = jnp.where(y >= 0.0, y, 0.01 * y)
    elif act == 'elu':
        y = jnp.where(y >= 0.0, y, jnp.expm1(y))
    o_ref[...] = y


def _mm(a, b, bias=None, act=None, tile_m=512):
    m, k = a.shape
    _, n = b.shape
    kp = _rup(k, 128)
    np_ = _rup(n, 128)
    tile_m = min(tile_m, _rup(m, 8))
    mp = _rup(m, tile_m)
    a = jnp.pad(a, ((0, mp - m), (0, kp - k)))
    b = jnp.pad(b, ((0, kp - k), (0, np_ - n)))
    has_bias = bias is not None
    operands = [a, b]
    in_specs = [
        pl.BlockSpec((tile_m, kp), lambda i: (i, 0)),
        pl.BlockSpec((kp, np_), lambda i: (0, 0)),
    ]
    if has_bias:
        operands.append(jnp.pad(bias.reshape(1, -1), ((0, 0), (0, np_ - n))))
        in_specs.append(pl.BlockSpec((1, np_), lambda i: (0, 0)))
    out = pl.pallas_call(
        functools.partial(_mm_body, act, has_bias),
        grid=(mp // tile_m,),
        in_specs=in_specs,
        out_specs=pl.BlockSpec((tile_m, np_), lambda i: (i, 0)),
        out_shape=jax.ShapeDtypeStruct((mp, np_), jnp.float32),
    )(*operands)
    return out[:m, :n]


def _gru_body(x_ref, h_ref, wi_ref, wh_ref, bi_ref, bh_ref, o_ref):
    gi = jnp.dot(x_ref[...], wi_ref[...],
                 preferred_element_type=jnp.float32) + bi_ref[...]
    gh = jnp.dot(h_ref[...], wh_ref[...],
                 preferred_element_type=jnp.float32) + bh_ref[...]
    d = _HID
    r = jax.nn.sigmoid(gi[:, :d] + gh[:, :d])
    z = jax.nn.sigmoid(gi[:, d:2 * d] + gh[:, d:2 * d])
    n = jnp.tanh(gi[:, 2 * d:] + r * gh[:, 2 * d:])
    o_ref[...] = (1.0 - z) * n + z * h_ref[...]


def _gru(xin, h, wi, wh, bi, bh, tile_m=512):
    m = xin.shape[0]
    tile_m = min(tile_m, _rup(m, 8))
    mp = _rup(m, tile_m)
    xp = jnp.pad(xin, ((0, mp - m), (0, 0)))
    hp = jnp.pad(h, ((0, mp - m), (0, 0)))
    out = pl.pallas_call(
        _gru_body,
        grid=(mp // tile_m,),
        in_specs=[
            pl.BlockSpec((tile_m, _HID), lambda i: (i, 0)),
            pl.BlockSpec((tile_m, _HID), lambda i: (i, 0)),
            pl.BlockSpec((_HID, 3 * _HID), lambda i: (0, 0)),
            pl.BlockSpec((_HID, 3 * _HID), lambda i: (0, 0)),
            pl.BlockSpec((1, 3 * _HID), lambda i: (0, 0)),
            pl.BlockSpec((1, 3 * _HID), lambda i: (0, 0)),
        ],
        out_specs=pl.BlockSpec((tile_m, _HID), lambda i: (i, 0)),
        out_shape=jax.ShapeDtypeStruct((mp, _HID), jnp.float32),
    )(xp, hp, wi, wh, bi.reshape(1, -1), bh.reshape(1, -1))
    return out[:m]


def _ge_body(hs_ref, ea_ref, wa_ref, wb_ref, b_ref, w2_ref, al_ref,
             xw2_ref, xal_ref):
    y = jnp.dot(hs_ref[...], wa_ref[...], preferred_element_type=jnp.float32)
    y = y + jnp.dot(ea_ref[...], wb_ref[...],
                    preferred_element_type=jnp.float32) + b_ref[...]
    xj = jnp.where(y >= 0.0, y, 0.01 * y)
    xw2_ref[...] = jnp.dot(xj, w2_ref[...],
                           preferred_element_type=jnp.float32)
    xal_ref[...] = jnp.dot(xj, al_ref[...],
                           preferred_element_type=jnp.float32)


def _ge_chain(hsrc, ea, ge_w, ge_b, ge_w2, ge_al, tile_m=512):
    e, d = hsrc.shape
    de = ea.shape[1]
    dep = _rup(de, 128)
    mp = _rup(e, tile_m)
    hsrc = jnp.pad(hsrc, ((0, mp - e), (0, 0)))
    ea = jnp.pad(ea, ((0, mp - e), (0, dep - de)))
    wa = ge_w[:d]
    wb = jnp.pad(ge_w[d:], ((0, dep - de), (0, 0)))
    al = jnp.pad(ge_al[:, None], ((0, 0), (0, 127)))
    xw2, xal = pl.pallas_call(
        _ge_body,
        grid=(mp // tile_m,),
        in_specs=[
            pl.BlockSpec((tile_m, d), lambda i: (i, 0)),
            pl.BlockSpec((tile_m, dep), lambda i: (i, 0)),
            pl.BlockSpec((d, _HID), lambda i: (0, 0)),
            pl.BlockSpec((dep, _HID), lambda i: (0, 0)),
            pl.BlockSpec((1, _HID), lambda i: (0, 0)),
            pl.BlockSpec((_HID, _HID), lambda i: (0, 0)),
            pl.BlockSpec((_HID, 128), lambda i: (0, 0)),
        ],
        out_specs=[
            pl.BlockSpec((tile_m, _HID), lambda i: (i, 0)),
            pl.BlockSpec((tile_m, 128), lambda i: (i, 0)),
        ],
        out_shape=[
            jax.ShapeDtypeStruct((mp, _HID), jnp.float32),
            jax.ShapeDtypeStruct((mp, 128), jnp.float32),
        ],
    )(hsrc, ea, wa, wb, ge_b.reshape(1, -1), ge_w2, al)
    return xw2[:e], xal[:e, 0]


def _seg_softmax(logits, segs, num):
    m = jax.ops.segment_max(logits, segs, num_segments=num)
    m = jnp.where(jnp.isfinite(m), m, 0.0)
    ex = jnp.exp(logits - m[segs])
    den = jax.ops.segment_sum(ex, segs, num_segments=num)
    return ex / (den[segs] + 1e-16)


def _head(x, edge_attr, src, dst, batch, p, n_nodes, n_graphs):
    h = _mm(x, p['lin1_W'], bias=p['lin1_b'], act='leaky')
    xw2, xal = _ge_chain(h[src], edge_attr, p['ge_W'], p['ge_b'],
                         p['ge_W2'], p['ge_al'])
    har = _mm(h, p['ge_ar'][:, None])[:, 0]
    alpha = jax.nn.leaky_relu(xal + har[dst])
    alpha = _seg_softmax(alpha, dst, n_nodes)
    msg = jax.ops.segment_sum(alpha[:, None] * xw2, dst, num_segments=n_nodes)
    msg = jax.nn.elu(msg)
    h = _gru(msg, h, p['gru1_Wi'], p['gru1_Wh'], p['gru1_bi'], p['gru1_bh'])

    hw = _mm(h, p['gat_W'])
    asd = _mm(hw, jnp.stack([p['gat_as'], p['gat_ad']], axis=1))
    alpha = jax.nn.leaky_relu(asd[src, 0] + asd[dst, 1])
    alpha = _seg_softmax(alpha, dst, n_nodes)
    msg = jax.ops.segment_sum(alpha[:, None] * hw[src], dst,
                              num_segments=n_nodes)
    msg = jax.nn.elu(msg)
    h = _gru(msg, h, p['gru2_Wi'], p['gru2_Wh'], p['gru2_bi'], p['gru2_bh'])

    g = jax.nn.relu(jax.ops.segment_sum(h, batch, num_segments=n_graphs))
    hw = _mm(h, p['mol_W'])
    hw_as = _mm(hw, p['mol_as'][:, None])[:, 0]
    for _ in range(_L_MOL):
        g_ad = _mm(g, p['mol_ad'][:, None])[:, 0]
        alpha = jax.nn.leaky_relu(hw_as + g_ad[batch])
        alpha = _seg_softmax(alpha, batch, n_graphs)
        msg = jax.ops.segment_sum(alpha[:, None] * hw, batch,
                                  num_segments=n_graphs)
        msg = jax.nn.elu(msg)
        g = _gru(msg, g, p['grum_Wi'], p['grum_Wh'], p['grum_bi'],
                 p['grum_bh'])
    return _mm(g, p['lin2_W'], bias=p['lin2_b'])


_HEAD_KEYS = ('lin1', 'ge', 'gru', 'gat', 'mol', 'lin2')


@jax.jit
def _forward_impl(x, edge_index, edge_attr, batch, params):
    n_nodes = x.shape[0]
    n_graphs = params['fa_W'].shape[1]
    src, dst = edge_index[0], edge_index[1]
    outs = []
    for hi in range(_NUM_HEADS):
        ph = {k: v[hi] for k, v in params.items()
              if k.startswith(_HEAD_KEYS)}
        outs.append(_head(x, edge_attr, src, dst, batch, ph, n_nodes, 256))
    cat = jnp.concatenate(outs, axis=-1)
    y = _mm(cat, params['fa_W'], bias=params['fa_b'])
    mu = y.mean(axis=0)
    var = y.var(axis=0)
    y = (y - mu) / jnp.sqrt(var + 1e-5) * params['fa_gamma'] + params['fa_beta']
    return jax.nn.relu(y)


def kernel(x, edge_index, edge_attr, batch, params):
    return _forward_impl(x, edge_index, edge_attr, batch, params)
